# Initial kernel scaffold; baseline (speedup 1.0000x reference)
import jax
import jax.numpy as jnp
from jax.experimental import pallas as pl


def kernel(x, edge_index, Wz, bz, Wr, br, Wh, bh, Wlz, blz, Wlr, blr, Wlh, blh, att, bn_g, bn_b, W1, b1, W2, b2, W3, b3):
    raise NotImplementedError("write your pallas kernel here")


# speedup vs baseline: 116.9573x; 116.9573x over previous
import jax
import jax.numpy as jnp
from jax.experimental import pallas as pl
from functools import partial

N_NODES = 18
PERIODS = 256
OUT = 64
E_RAW = 162
E = E_RAW + N_NODES

BB = 16
TCHUNK = 32


def _leaky(v):
    return jnp.where(v >= 0, v, 0.01 * v)


def _fused_kernel(x_ref, ei_ref, Wz_ref, bz_ref, Wh_ref, bh_ref,
                  Wlz_ref, blz_ref, Wlh_ref, blh_ref, att_ref,
                  bng_ref, bnb_ref, W1_ref, b1_ref, W2_ref, b2_ref,
                  W3_ref, b3_ref, out_ref):
    f32 = jnp.float32

    ei = ei_ref[...]
    loop_iota = jax.lax.broadcasted_iota(jnp.int32, (1, N_NODES), 1)
    e_src = jnp.concatenate([ei[0:1, :], loop_iota], axis=1)
    e_dst = jnp.concatenate([ei[1:2, :], loop_iota], axis=1)
    ncol = jax.lax.broadcasted_iota(jnp.int32, (N_NODES, 1), 0)
    St = (e_src == ncol).astype(f32)
    Dt = (e_dst == ncol).astype(f32)
    deg = jnp.sum(Dt, axis=1, keepdims=True)
    dinv = jnp.where(deg > 0, jax.lax.rsqrt(deg), 0.0)
    dsrc = jnp.sum(dinv * St, axis=0, keepdims=True)
    ddst = jnp.sum(dinv * Dt, axis=0, keepdims=True)
    norm = dsrc * ddst
    A = jax.lax.dot_general(Dt, St * norm,
                            (((1,), (1,)), ((), ())),
                            preferred_element_type=f32, precision=jax.lax.Precision.HIGHEST)

    Wlz_top = Wlz_ref[0:OUT, :]
    Wlh_top = Wlh_ref[0:OUT, :]
    uz = jnp.dot(Wz_ref[...], Wlz_top, preferred_element_type=f32, precision=jax.lax.Precision.HIGHEST)
    cz = jnp.dot(bz_ref[...], Wlz_top, preferred_element_type=f32, precision=jax.lax.Precision.HIGHEST) + blz_ref[...]
    uh = jnp.dot(Wh_ref[...], Wlh_top, preferred_element_type=f32, precision=jax.lax.Precision.HIGHEST)
    ch = jnp.dot(bh_ref[...], Wlh_top, preferred_element_type=f32, precision=jax.lax.Precision.HIGHEST) + blh_ref[...]

    att = att_ref[...]
    att = att - jnp.max(att, axis=1, keepdims=True)
    p = jnp.exp(att)
    probs = p / jnp.sum(p, axis=1, keepdims=True)

    xb = x_ref[...]
    Y = jax.lax.dot_general(A, xb, (((1,), (1,)), ((), ())),
                            preferred_element_type=f32, precision=jax.lax.Precision.HIGHEST)
    R = N_NODES * BB
    Yf = Y.reshape(R, PERIODS)

    uz3 = uz[:, None, :]
    cz3 = cz[:, None, :]
    uh3 = uh[:, None, :]
    ch3 = ch[:, None, :]
    acc = jnp.zeros((R, OUT), f32)
    for c in range(PERIODS // TCHUNK):
        Yc = Yf[:, c * TCHUNK:(c + 1) * TCHUNK][:, :, None]
        pc = probs[:, c * TCHUNK:(c + 1) * TCHUNK][0][None, :, None]
        g = jax.nn.sigmoid(-(Yc * uz3 + cz3)) * jnp.tanh(Yc * uh3 + ch3)
        acc = acc + jnp.sum(g * pc, axis=1)

    h = acc.reshape(N_NODES, BB, OUT)
    scale = 1.0 / jnp.sqrt(jnp.float32(1.0 + 1e-5))
    h = h * (scale * bng_ref[...][:, :, None]) + bnb_ref[...][:, :, None]
    h = _leaky(h)

    o1 = jax.lax.dot_general(h, W1_ref[...],
                             (((2,), (1,)), ((0,), (0,))),
                             preferred_element_type=f32, precision=jax.lax.Precision.HIGHEST)
    h1 = _leaky(jnp.sum(o1, axis=0) + b1_ref[...])
    h2 = _leaky(jnp.dot(h1, W2_ref[...], preferred_element_type=f32, precision=jax.lax.Precision.HIGHEST) + b2_ref[...])
    h3 = jnp.dot(h2, W3_ref[...], preferred_element_type=f32, precision=jax.lax.Precision.HIGHEST) + b3_ref[...]
    out_ref[...] = h3


@jax.jit
def kernel(x, edge_index, Wz, bz, Wr, br, Wh, bh, Wlz, blz, Wlr, blr,
           Wlh, blh, att, bn_g, bn_b, W1, b1, W2, b2, W3, b3):
    B = x.shape[0]
    xs = x.reshape(B, N_NODES, PERIODS)
    grid = (B // BB,)

    full = lambda *s: pl.BlockSpec(s, lambda i: (0,) * len(s))
    out = pl.pallas_call(
        _fused_kernel,
        grid=grid,
        in_specs=[
            pl.BlockSpec((BB, N_NODES, PERIODS), lambda i: (i, 0, 0)),
            full(2, E_RAW),
            full(1, OUT), full(1, OUT),
            full(1, OUT), full(1, OUT),
            full(2 * OUT, OUT), full(1, OUT),
            full(2 * OUT, OUT), full(1, OUT),
            full(1, PERIODS),
            full(N_NODES, 1), full(N_NODES, 1),
            full(N_NODES, OUT, OUT), full(1, OUT),
            full(OUT, 32), full(1, 32),
            full(32, 1), full(1, 1),
        ],
        out_specs=pl.BlockSpec((BB, 1), lambda i: (i, 0)),
        out_shape=jax.ShapeDtypeStruct((B, 1), jnp.float32),
    )(xs, edge_index, Wz, bz.reshape(1, OUT), Wh, bh.reshape(1, OUT),
      Wlz, blz.reshape(1, OUT), Wlh, blh.reshape(1, OUT),
      att.reshape(1, PERIODS), bn_g.reshape(N_NODES, 1), bn_b.reshape(N_NODES, 1),
      W1.reshape(N_NODES, OUT, OUT), b1.reshape(1, OUT),
      W2, b2.reshape(1, 32), W3, b3.reshape(1, 1))
    return out.reshape(B)
